# baseline (speedup 1.0000x reference)
import jax
import jax.numpy as jnp
from jax import lax
from jax.experimental import pallas as pl
from jax.experimental.pallas import tpu as pltpu
from jax.experimental.pallas import tpu_sc as plsc

NA = 50000
D = 128
RU = 32

NC = 2
NS = 16
NW = NC * NS

BOND = (50000, 2, 1600, 400, 4)
ANGLE = (60000, 3, 1920, 480, 4)
TORSION = (70000, 4, 2240, 448, 5)
MAXROWS = 1792
P14 = (100000, 3200, 640, 5)
PNB = (200000, 6400, 640, 10)
PBLK = 640

_mesh = plsc.VectorSubcoreMesh(
    core_axis_name="c", subcore_axis_name="s", num_cores=NC, num_subcores=NS)


def _worker_id():
  return lax.axis_index("s") * NC + lax.axis_index("c")


def _stage_a_body(h_ref, w1_ref, b1_ref, w2_ref, b2_ref,
                  zb_ref, za_ref, zt_ref, ke_ref):
  z = jnp.dot(h_ref[...], w1_ref[...], preferred_element_type=jnp.float32)
  zb_ref[...] = z[:, RU:2 * RU]
  za_ref[...] = z[:, 2 * RU:3 * RU]
  zt_ref[...] = z[:, 3 * RU:4 * RU]
  t = jnp.tanh(z[:, :RU] + b1_ref[...])
  ke = jnp.abs(jnp.dot(t, w2_ref[...], preferred_element_type=jnp.float32)
               + b2_ref[...])
  sq = jnp.sqrt(ke[:, 1:2])
  ke_ref[...] = jnp.concatenate([ke, sq, sq], axis=1)


def _stage_a(h, w1cat, b1a, w2a, b2a):
  rb = 1000
  grid = (NA // rb,)
  return pl.pallas_call(
      _stage_a_body,
      grid=grid,
      in_specs=[
          pl.BlockSpec((rb, D), lambda i: (i, 0)),
          pl.BlockSpec((D, 4 * RU), lambda i: (0, 0)),
          pl.BlockSpec((1, RU), lambda i: (0, 0)),
          pl.BlockSpec((RU, 2), lambda i: (0, 0)),
          pl.BlockSpec((1, 2), lambda i: (0, 0)),
      ],
      out_specs=[
          pl.BlockSpec((rb, RU), lambda i: (i, 0)),
          pl.BlockSpec((rb, RU), lambda i: (i, 0)),
          pl.BlockSpec((rb, RU), lambda i: (i, 0)),
          pl.BlockSpec((rb, 4), lambda i: (i, 0)),
      ],
      out_shape=[
          jax.ShapeDtypeStruct((NA, RU), jnp.float32),
          jax.ShapeDtypeStruct((NA, RU), jnp.float32),
          jax.ShapeDtypeStruct((NA, RU), jnp.float32),
          jax.ShapeDtypeStruct((NA, 4), jnp.float32),
      ],
  )(h, w1cat, b1a, w2a, b2a)


def _rel_pump(z_hbm, idx_hbm, out_hbm, spec, ibufs, rbufs, gsems, ssems):
  _, arity, _, blk, nblk = spec
  ab = arity * blk
  wb0 = _worker_id() * nblk

  def idx_load(b):
    off = pl.multiple_of((wb0 + b) * ab, 8)
    pltpu.sync_copy(idx_hbm.at[pl.ds(off, ab)],
                    ibufs[b % 2].at[pl.ds(0, ab)])

  def gather_start(b):
    return pltpu.async_copy(z_hbm.at[ibufs[b % 2].at[pl.ds(0, ab)]],
                            rbufs[b % 2].at[pl.ds(0, ab)], gsems[b % 2])

  def store_start(b):
    off = pl.multiple_of((wb0 + b) * ab, 8)
    return pltpu.async_copy(rbufs[b % 2].at[pl.ds(0, ab)],
                            out_hbm.at[pl.ds(off, ab)], ssems[b % 2])

  gathers = {}
  stores = {}
  idx_load(0)
  gathers[0] = gather_start(0)
  if nblk > 1:
    idx_load(1)
    gathers[1] = gather_start(1)
  for b in range(nblk):
    gathers.pop(b).wait()
    stores[b] = store_start(b)
    if b + 2 < nblk:
      idx_load(b + 2)
      stores.pop(b).wait()
      gathers[b + 2] = gather_start(b + 2)
  for s in stores.values():
    s.wait()


def _relations_body(zb, za, zt, ib, ia, it,
                    sb, sa, st,
                    i0, i1, r0, r1, g0, g1, s0, s1):
  _rel_pump(zb, ib, sb, BOND, (i0, i1), (r0, r1), (g0, g1), (s0, s1))
  _rel_pump(za, ia, sa, ANGLE, (i0, i1), (r0, r1), (g0, g1), (s0, s1))
  _rel_pump(zt, it, st, TORSION, (i0, i1), (r0, r1), (g0, g1), (s0, s1))


def _relations(zb, za, zt, idx_flat):
  out_type = [
      jax.ShapeDtypeStruct((BOND[1] * BOND[2] * NW, RU), jnp.float32),
      jax.ShapeDtypeStruct((ANGLE[1] * ANGLE[2] * NW, RU), jnp.float32),
      jax.ShapeDtypeStruct((TORSION[1] * TORSION[2] * NW, RU), jnp.float32),
  ]
  scratch = [
      pltpu.VMEM((MAXROWS,), jnp.int32),
      pltpu.VMEM((MAXROWS,), jnp.int32),
      pltpu.VMEM((MAXROWS, RU), jnp.float32),
      pltpu.VMEM((MAXROWS, RU), jnp.float32),
      pltpu.SemaphoreType.DMA,
      pltpu.SemaphoreType.DMA,
      pltpu.SemaphoreType.DMA,
      pltpu.SemaphoreType.DMA,
  ]
  fn = pl.kernel(_relations_body, out_type=out_type, mesh=_mesh,
                 scratch_types=scratch,
                 compiler_params=pltpu.CompilerParams(
                     use_tc_tiling_on_sc=False))
  return fn(zb, za, zt, *idx_flat)


def _pair_loop(i0_hbm, i1_hbm, sig_hbm, eps_hbm, spec,
               ktbl_v, stbl_v, ia_v, ib_v, so_v, eo_v):
  _, chunk, blk, nblk = spec
  base = _worker_id() * chunk

  def group(si, _):
    o = pl.multiple_of(si * 16, 16)
    ia = ia_v[pl.ds(o, 16)]
    ib = ib_v[pl.ds(o, 16)]
    ka = plsc.load_gather(ktbl_v, [ia])
    kb = plsc.load_gather(ktbl_v, [ib])
    so_v[pl.ds(o, 16)] = (ka + kb) * 0.5
    ea = plsc.load_gather(stbl_v, [ia])
    eb = plsc.load_gather(stbl_v, [ib])
    eo_v[pl.ds(o, 16)] = ea * eb
    return 0

  for b in range(nblk):
    off = pl.multiple_of(base + b * blk, 8)
    pltpu.sync_copy(i0_hbm.at[pl.ds(off, blk)], ia_v)
    pltpu.sync_copy(i1_hbm.at[pl.ds(off, blk)], ib_v)
    lax.fori_loop(0, blk // 16, group, 0)
    pltpu.sync_copy(so_v, sig_hbm.at[pl.ds(off, blk)])
    pltpu.sync_copy(eo_v, eps_hbm.at[pl.ds(off, blk)])


def _pairs_body(ktbl, stbl, p140, p141, pnb0, pnb1,
                sig14, eps14, signb, epsnb,
                ktbl_v, stbl_v, ia_v, ib_v, so_v, eo_v):
  pltpu.sync_copy(ktbl, ktbl_v)
  pltpu.sync_copy(stbl, stbl_v)
  _pair_loop(p140, p141, sig14, eps14, P14,
             ktbl_v, stbl_v, ia_v, ib_v, so_v, eo_v)
  _pair_loop(pnb0, pnb1, signb, epsnb, PNB,
             ktbl_v, stbl_v, ia_v, ib_v, so_v, eo_v)


def _pairs(ktbl, stbl, p140, p141, pnb0, pnb1):
  out_type = [
      jax.ShapeDtypeStruct((P14[1] * NW,), jnp.float32),
      jax.ShapeDtypeStruct((P14[1] * NW,), jnp.float32),
      jax.ShapeDtypeStruct((PNB[1] * NW,), jnp.float32),
      jax.ShapeDtypeStruct((PNB[1] * NW,), jnp.float32),
  ]
  scratch = [
      pltpu.VMEM((NA,), jnp.float32),
      pltpu.VMEM((NA,), jnp.float32),
      pltpu.VMEM((PBLK,), jnp.int32),
      pltpu.VMEM((PBLK,), jnp.int32),
      pltpu.VMEM((PBLK,), jnp.float32),
      pltpu.VMEM((PBLK,), jnp.float32),
  ]
  fn = pl.kernel(_pairs_body, out_type=out_type, mesh=_mesh,
                 scratch_types=scratch,
                 compiler_params=pltpu.CompilerParams(
                     use_tc_tiling_on_sc=False, needs_layout_passes=False))
  return fn(ktbl, stbl, p140, p141, pnb0, pnb1)


def _make_stage_c_body(arity):
  def body(*refs):
    s_refs = refs[:arity]
    b1_ref, w2_ref, b2_ref, out_ref = refs[arity:]
    s = s_refs[0][...]
    for r in s_refs[1:]:
      s = s + r[...]
    t = jnp.tanh(s + b1_ref[...])
    out_ref[...] = jnp.abs(
        jnp.dot(t, w2_ref[...], preferred_element_type=jnp.float32)
        + b2_ref[...])
  return body


def _stage_c(rows, spec, b1, w2, b2):
  _, arity, chunk, blk, nblk = spec
  npad = chunk * NW
  grid = (NW * nblk,)
  in_specs = [
      pl.BlockSpec((blk, RU), (lambda a: (lambda i: (i * arity + a, 0)))(a))
      for a in range(arity)
  ] + [
      pl.BlockSpec((1, RU), lambda i: (0, 0)),
      pl.BlockSpec((RU, 2), lambda i: (0, 0)),
      pl.BlockSpec((1, 2), lambda i: (0, 0)),
  ]
  return pl.pallas_call(
      _make_stage_c_body(arity),
      grid=grid,
      in_specs=in_specs,
      out_specs=pl.BlockSpec((blk, 2), lambda i: (i, 0)),
      out_shape=jax.ShapeDtypeStruct((npad, 2), jnp.float32),
  )(*([rows] * arity), b1, w2, b2)


def _flat_idx(atoms, spec):
  n, arity, chunk, blk, nblk = spec
  npad = chunk * NW
  cols = jnp.pad(atoms, ((0, npad - n), (0, 0)))
  arr = cols.T.reshape(arity, NW * nblk, blk)
  return arr.transpose(1, 0, 2).reshape(-1)


def _pad_cols(atoms, npad):
  n = atoms.shape[0]
  return [jnp.pad(atoms[:, a], (0, npad - n)) for a in range(atoms.shape[1])]


def kernel(h, bond_atoms, angle_atoms, torsion_atoms, one_four_atoms,
           nonbonded_atoms,
           W1_atom, b1_atom, W2_atom, b2_atom,
           W1_bond, b1_bond, W2_bond, b2_bond,
           W1_angle, b1_angle, W2_angle, b2_angle,
           W1_torsion, b1_torsion, W2_torsion, b2_torsion):
  w1cat = jnp.concatenate([W1_atom, W1_bond, W1_angle, W1_torsion], axis=1)
  zb, za, zt, ke = _stage_a(h, w1cat, b1_atom.reshape(1, RU),
                            W2_atom, b2_atom.reshape(1, 2))
  k_atom = ke[:, 0]
  eq_atom = ke[:, 1]
  sq_eq = ke[:, 2]

  idx_flat = (_flat_idx(bond_atoms, BOND),
              _flat_idx(angle_atoms, ANGLE),
              _flat_idx(torsion_atoms, TORSION))
  sb, sa, st = _relations(zb, za, zt, idx_flat)

  p14 = _pad_cols(one_four_atoms, P14[1] * NW)
  pnb = _pad_cols(nonbonded_atoms, PNB[1] * NW)
  sig14, eps14, signb, epsnb = _pairs(k_atom, sq_eq, p14[0], p14[1],
                                      pnb[0], pnb[1])

  rb_ = _stage_c(sb, BOND, b1_bond.reshape(1, RU), W2_bond,
                 b2_bond.reshape(1, 2))
  ra_ = _stage_c(sa, ANGLE, b1_angle.reshape(1, RU), W2_angle,
                 b2_angle.reshape(1, 2))
  rt_ = _stage_c(st, TORSION, b1_torsion.reshape(1, RU), W2_torsion,
                 b2_torsion.reshape(1, 2))

  return jnp.concatenate([
      k_atom, eq_atom,
      rb_[:BOND[0], 0], rb_[:BOND[0], 1],
      ra_[:ANGLE[0], 0], ra_[:ANGLE[0], 1],
      rt_[:TORSION[0], 0], rt_[:TORSION[0], 1],
      sig14[:P14[0]], eps14[:P14[0]],
      signb[:PNB[0]], epsnb[:PNB[0]],
  ])


# speedup vs baseline: 1.5231x; 1.5231x over previous
import jax
import jax.numpy as jnp
from jax import lax
from jax.experimental import pallas as pl
from jax.experimental.pallas import tpu as pltpu
from jax.experimental.pallas import tpu_sc as plsc

NA = 50000
NZ = 50176
D = 128
RU = 32

NC = 2
NS = 16
NW = NC * NS

BOND = (50000, 2, 1664, 416, 4)
ANGLE = (60000, 3, 1920, 384, 5)
TORSION = (70000, 4, 2240, 320, 7)
MAXROWS = 1280
MAXB4 = 104
P14 = (100000, 3200, 640, 5)
PNB = (200000, 6400, 640, 10)
PBLK = 640

_mesh = plsc.VectorSubcoreMesh(
    core_axis_name="c", subcore_axis_name="s", num_cores=NC, num_subcores=NS)


def _worker_id():
  return lax.axis_index("s") * NC + lax.axis_index("c")


def _stage_a_body(h_ref, w1_ref, b1_ref, w2_ref, b2_ref,
                  zb_ref, za_ref, zt_ref, k_ref, e_ref, s_ref,
                  zscr_b, zscr_a, zscr_t):
  rb = h_ref.shape[0]
  z = jnp.dot(h_ref[...], w1_ref[...], preferred_element_type=jnp.float32)
  for i, (scr, ref) in ((1, (zscr_b, zb_ref)), (2, (zscr_a, za_ref)),
                        (3, (zscr_t, zt_ref))):
    scr[...] = z[:, RU * i:RU * (i + 1)]
    ref[...] = jnp.concatenate(
        [scr[pl.ds(c, rb // 4, 4), :] for c in range(4)], axis=1)
  t = jnp.tanh(z[:, :RU] + b1_ref[...])
  ke = jnp.abs(jnp.dot(t, w2_ref[...], preferred_element_type=jnp.float32)
               + b2_ref[...])
  k_ref[...] = ke[:, 0]
  e_ref[...] = ke[:, 1]
  s_ref[...] = jnp.sqrt(ke[:, 1])


def _stage_a(h, w1cat, b1a, w2a, b2a):
  rb = 1024
  grid = (pl.cdiv(NA, rb),)
  return pl.pallas_call(
      _stage_a_body,
      grid=grid,
      in_specs=[
          pl.BlockSpec((rb, D), lambda i: (i, 0)),
          pl.BlockSpec((D, 4 * RU), lambda i: (0, 0)),
          pl.BlockSpec((1, RU), lambda i: (0, 0)),
          pl.BlockSpec((RU, 2), lambda i: (0, 0)),
          pl.BlockSpec((1, 2), lambda i: (0, 0)),
      ],
      out_specs=[
          pl.BlockSpec((rb // 4, D), lambda i: (i, 0)),
          pl.BlockSpec((rb // 4, D), lambda i: (i, 0)),
          pl.BlockSpec((rb // 4, D), lambda i: (i, 0)),
          pl.BlockSpec((rb,), lambda i: (i,)),
          pl.BlockSpec((rb,), lambda i: (i,)),
          pl.BlockSpec((rb,), lambda i: (i,)),
      ],
      out_shape=[
          jax.ShapeDtypeStruct((NZ // 4, D), jnp.float32),
          jax.ShapeDtypeStruct((NZ // 4, D), jnp.float32),
          jax.ShapeDtypeStruct((NZ // 4, D), jnp.float32),
          jax.ShapeDtypeStruct((NA,), jnp.float32),
          jax.ShapeDtypeStruct((NA,), jnp.float32),
          jax.ShapeDtypeStruct((NA,), jnp.float32),
      ],
      scratch_shapes=[pltpu.VMEM((rb, RU), jnp.float32),
                      pltpu.VMEM((rb, RU), jnp.float32),
                      pltpu.VMEM((rb, RU), jnp.float32)],
  )(h, w1cat, b1a, w2a, b2a)


def _rel_pump(z_hbm, idx_hbm, out_hbm, spec,
              ibufs, rbufs, sbuf, gsems, ssem):
  _, arity, _, blk, nblk = spec
  ab = arity * blk
  b4 = blk // 4
  wb0 = _worker_id() * nblk

  def idx_load(b):
    off = pl.multiple_of((wb0 + b) * ab, 8)
    pltpu.sync_copy(idx_hbm.at[pl.ds(off, ab)],
                    ibufs[b % 2].at[pl.ds(0, ab)])

  def gather_start(b):
    return pltpu.async_copy(z_hbm.at[ibufs[b % 2].at[pl.ds(0, ab)]],
                            rbufs[b % 2].at[pl.ds(0, ab)], gsems[b % 2])

  def sum_block(b):
    rbuf = rbufs[b % 2]

    def body(p, _):
      for c in range(4):
        e = 4 * p + c
        for h in range(2):
          acc = rbuf[e, pl.ds(16 * h, 16)]
          for a in range(1, arity):
            acc = acc + rbuf[a * blk + e, pl.ds(16 * h, 16)]
          sbuf[p, pl.ds(32 * c + 16 * h, 16)] = acc
      return 0

    lax.fori_loop(0, b4, body, 0)

  def store_start(b):
    off = pl.multiple_of((wb0 + b) * b4, 8)
    return pltpu.async_copy(sbuf.at[pl.ds(0, b4)],
                            out_hbm.at[pl.ds(off, b4)], ssem)

  idx_load(0)
  gathers = {0: gather_start(0)}
  if nblk > 1:
    idx_load(1)
  store = None
  for b in range(nblk):
    gathers.pop(b).wait()
    if b + 1 < nblk:
      gathers[b + 1] = gather_start(b + 1)
    if b + 2 < nblk:
      idx_load(b + 2)
    if store is not None:
      store.wait()
    sum_block(b)
    store = store_start(b)
  store.wait()


def _relations_body(zb, za, zt, ib, ia, it,
                    sb, sa, st,
                    i0, i1, r0, r1, sbuf, g0, g1, ssem):
  _rel_pump(zb, ib, sb, BOND, (i0, i1), (r0, r1), sbuf, (g0, g1), ssem)
  _rel_pump(za, ia, sa, ANGLE, (i0, i1), (r0, r1), sbuf, (g0, g1), ssem)
  _rel_pump(zt, it, st, TORSION, (i0, i1), (r0, r1), sbuf, (g0, g1), ssem)


def _relations(zb, za, zt, idx_flat):
  out_type = [
      jax.ShapeDtypeStruct((BOND[2] * NW // 4, D), jnp.float32),
      jax.ShapeDtypeStruct((ANGLE[2] * NW // 4, D), jnp.float32),
      jax.ShapeDtypeStruct((TORSION[2] * NW // 4, D), jnp.float32),
  ]
  scratch = [
      pltpu.VMEM((MAXROWS,), jnp.int32),
      pltpu.VMEM((MAXROWS,), jnp.int32),
      pltpu.VMEM((MAXROWS, RU), jnp.float32),
      pltpu.VMEM((MAXROWS, RU), jnp.float32),
      pltpu.VMEM((MAXB4, D), jnp.float32),
      pltpu.SemaphoreType.DMA,
      pltpu.SemaphoreType.DMA,
      pltpu.SemaphoreType.DMA,
  ]
  fn = pl.kernel(_relations_body, out_type=out_type, mesh=_mesh,
                 scratch_types=scratch,
                 compiler_params=pltpu.CompilerParams(
                     use_tc_tiling_on_sc=False))
  return fn(zb, za, zt, *idx_flat)


def _pair_loop(i0_hbm, i1_hbm, sig_hbm, eps_hbm, spec,
               ktbl_v, stbl_v, ia_v, ib_v, so_v, eo_v):
  _, chunk, blk, nblk = spec
  base = _worker_id() * chunk

  def group(si, _):
    o = pl.multiple_of(si * 16, 16)
    ia = ia_v[pl.ds(o, 16)]
    ib = ib_v[pl.ds(o, 16)]
    ka = plsc.load_gather(ktbl_v, [ia])
    kb = plsc.load_gather(ktbl_v, [ib])
    so_v[pl.ds(o, 16)] = (ka + kb) * 0.5
    ea = plsc.load_gather(stbl_v, [ia])
    eb = plsc.load_gather(stbl_v, [ib])
    eo_v[pl.ds(o, 16)] = ea * eb
    return 0

  for b in range(nblk):
    off = pl.multiple_of(base + b * blk, 8)
    pltpu.sync_copy(i0_hbm.at[pl.ds(off, blk)], ia_v)
    pltpu.sync_copy(i1_hbm.at[pl.ds(off, blk)], ib_v)
    lax.fori_loop(0, blk // 16, group, 0)
    pltpu.sync_copy(so_v, sig_hbm.at[pl.ds(off, blk)])
    pltpu.sync_copy(eo_v, eps_hbm.at[pl.ds(off, blk)])


def _pairs_body(ktbl, stbl, p140, p141, pnb0, pnb1,
                sig14, eps14, signb, epsnb,
                ktbl_v, stbl_v, ia_v, ib_v, so_v, eo_v):
  pltpu.sync_copy(ktbl, ktbl_v)
  pltpu.sync_copy(stbl, stbl_v)
  _pair_loop(p140, p141, sig14, eps14, P14,
             ktbl_v, stbl_v, ia_v, ib_v, so_v, eo_v)
  _pair_loop(pnb0, pnb1, signb, epsnb, PNB,
             ktbl_v, stbl_v, ia_v, ib_v, so_v, eo_v)


def _pairs(ktbl, stbl, p140, p141, pnb0, pnb1):
  out_type = [
      jax.ShapeDtypeStruct((P14[1] * NW,), jnp.float32),
      jax.ShapeDtypeStruct((P14[1] * NW,), jnp.float32),
      jax.ShapeDtypeStruct((PNB[1] * NW,), jnp.float32),
      jax.ShapeDtypeStruct((PNB[1] * NW,), jnp.float32),
  ]
  scratch = [
      pltpu.VMEM((NA,), jnp.float32),
      pltpu.VMEM((NA,), jnp.float32),
      pltpu.VMEM((PBLK,), jnp.int32),
      pltpu.VMEM((PBLK,), jnp.int32),
      pltpu.VMEM((PBLK,), jnp.float32),
      pltpu.VMEM((PBLK,), jnp.float32),
  ]
  fn = pl.kernel(_pairs_body, out_type=out_type, mesh=_mesh,
                 scratch_types=scratch,
                 compiler_params=pltpu.CompilerParams(
                     use_tc_tiling_on_sc=False, needs_layout_passes=False))
  return fn(ktbl, stbl, p140, p141, pnb0, pnb1)


def _stage_c_body(s_ref, b1_ref, wk_ref, we_ref, b2_ref, k_ref, e_ref):
  t = jnp.tanh(s_ref[...] + b1_ref[...])
  k_ref[...] = jnp.abs(
      jnp.dot(t, wk_ref[...], preferred_element_type=jnp.float32)
      + b2_ref[0, 0])
  e_ref[...] = jnp.abs(
      jnp.dot(t, we_ref[...], preferred_element_type=jnp.float32)
      + b2_ref[0, 1])


def _stage_c(rows4, spec, b1, w2, b2):
  _, _, chunk, _, _ = spec
  npad4 = chunk * NW // 4
  kblocks = npad4 // 128
  b1rep = jnp.tile(b1, 4).reshape(1, D)
  wk = jnp.kron(jnp.eye(4, dtype=jnp.float32), w2[:, 0:1])
  we = jnp.kron(jnp.eye(4, dtype=jnp.float32), w2[:, 1:2])
  k4, e4 = pl.pallas_call(
      _stage_c_body,
      grid=(kblocks,),
      in_specs=[
          pl.BlockSpec((128, D), lambda i: (i, 0)),
          pl.BlockSpec((1, D), lambda i: (0, 0)),
          pl.BlockSpec((D, 4), lambda i: (0, 0)),
          pl.BlockSpec((D, 4), lambda i: (0, 0)),
          pl.BlockSpec((1, 2), lambda i: (0, 0)),
      ],
      out_specs=[
          pl.BlockSpec((128, 4), lambda i: (i, 0)),
          pl.BlockSpec((128, 4), lambda i: (i, 0)),
      ],
      out_shape=[
          jax.ShapeDtypeStruct((npad4, 4), jnp.float32),
          jax.ShapeDtypeStruct((npad4, 4), jnp.float32),
      ],
  )(rows4, b1rep, wk, we, b2.reshape(1, 2))
  return k4.reshape(-1), e4.reshape(-1)


def _flat_idx(atoms, spec):
  n, arity, chunk, blk, nblk = spec
  npad = chunk * NW
  cols = jnp.pad(atoms, ((0, npad - n), (0, 0)))
  arr = cols.T.reshape(arity, NW * nblk, blk)
  return arr.transpose(1, 0, 2).reshape(-1)


def _pad_cols(atoms, npad):
  n = atoms.shape[0]
  return [jnp.pad(atoms[:, a], (0, npad - n)) for a in range(atoms.shape[1])]


def kernel(h, bond_atoms, angle_atoms, torsion_atoms, one_four_atoms,
           nonbonded_atoms,
           W1_atom, b1_atom, W2_atom, b2_atom,
           W1_bond, b1_bond, W2_bond, b2_bond,
           W1_angle, b1_angle, W2_angle, b2_angle,
           W1_torsion, b1_torsion, W2_torsion, b2_torsion):
  w1cat = jnp.concatenate([W1_atom, W1_bond, W1_angle, W1_torsion], axis=1)
  zb, za, zt, k_atom, eq_atom, sq_eq = _stage_a(
      h, w1cat, b1_atom.reshape(1, RU), W2_atom, b2_atom.reshape(1, 2))

  idx_flat = (_flat_idx(bond_atoms, BOND),
              _flat_idx(angle_atoms, ANGLE),
              _flat_idx(torsion_atoms, TORSION))
  sb, sa, st = _relations(zb.reshape(NZ, RU), za.reshape(NZ, RU),
                          zt.reshape(NZ, RU), idx_flat)

  p14 = _pad_cols(one_four_atoms, P14[1] * NW)
  pnb = _pad_cols(nonbonded_atoms, PNB[1] * NW)
  sig14, eps14, signb, epsnb = _pairs(k_atom, sq_eq, p14[0], p14[1],
                                      pnb[0], pnb[1])

  kb_, eb_ = _stage_c(sb, BOND, b1_bond, W2_bond, b2_bond)
  ka_, ea_ = _stage_c(sa, ANGLE, b1_angle, W2_angle, b2_angle)
  kt_, et_ = _stage_c(st, TORSION, b1_torsion, W2_torsion, b2_torsion)

  return jnp.concatenate([
      k_atom, eq_atom,
      kb_[:BOND[0]], eb_[:BOND[0]],
      ka_[:ANGLE[0]], ea_[:ANGLE[0]],
      kt_[:TORSION[0]], et_[:TORSION[0]],
      sig14[:P14[0]], eps14[:P14[0]],
      signb[:PNB[0]], epsnb[:PNB[0]],
  ])


# speedup vs baseline: 1.9918x; 1.3077x over previous
import jax
import jax.numpy as jnp
from jax import lax
from jax.experimental import pallas as pl
from jax.experimental.pallas import tpu as pltpu
from jax.experimental.pallas import tpu_sc as plsc

NA = 50000
NZ = 50176
D = 128
RU = 32

NC = 2
NS = 16
NW = NC * NS

BOND = (50000, 2, 1664, 416, 4)
ANGLE = (60000, 3, 1920, 384, 5)
TORSION = (70000, 4, 2240, 320, 7)
MAXROWS = 1280
MAXB4 = 104
P14 = (100000, 3200, 640, 5)
PNB = (200000, 6400, 640, 10)
PBLK = 640

_mesh = plsc.VectorSubcoreMesh(
    core_axis_name="c", subcore_axis_name="s", num_cores=NC, num_subcores=NS)


def _worker_id():
  return lax.axis_index("s") * NC + lax.axis_index("c")


def _stage_a_body(h_ref, w1_ref, b1_ref, w2_ref, b2_ref,
                  zb_ref, za_ref, zt_ref, k_ref, e_ref, s_ref,
                  zscr_b, zscr_a, zscr_t):
  rb = h_ref.shape[0]
  z = jnp.dot(h_ref[...], w1_ref[...], preferred_element_type=jnp.float32)
  for i, (scr, ref) in ((1, (zscr_b, zb_ref)), (2, (zscr_a, za_ref)),
                        (3, (zscr_t, zt_ref))):
    scr[...] = z[:, RU * i:RU * (i + 1)]
    ref[...] = jnp.concatenate(
        [scr[pl.ds(c, rb // 4, 4), :] for c in range(4)], axis=1)
  t = jnp.tanh(z[:, :RU] + b1_ref[...])
  ke = jnp.abs(jnp.dot(t, w2_ref[...], preferred_element_type=jnp.float32)
               + b2_ref[...])
  k_ref[...] = ke[:, 0]
  e_ref[...] = ke[:, 1]
  s_ref[...] = jnp.sqrt(ke[:, 1])


def _stage_a(h, w1cat, b1a, w2a, b2a):
  rb = 1024
  grid = (pl.cdiv(NA, rb),)
  return pl.pallas_call(
      _stage_a_body,
      grid=grid,
      in_specs=[
          pl.BlockSpec((rb, D), lambda i: (i, 0)),
          pl.BlockSpec((D, 4 * RU), lambda i: (0, 0)),
          pl.BlockSpec((1, RU), lambda i: (0, 0)),
          pl.BlockSpec((RU, 2), lambda i: (0, 0)),
          pl.BlockSpec((1, 2), lambda i: (0, 0)),
      ],
      out_specs=[
          pl.BlockSpec((rb // 4, D), lambda i: (i, 0)),
          pl.BlockSpec((rb // 4, D), lambda i: (i, 0)),
          pl.BlockSpec((rb // 4, D), lambda i: (i, 0)),
          pl.BlockSpec((rb,), lambda i: (i,)),
          pl.BlockSpec((rb,), lambda i: (i,)),
          pl.BlockSpec((rb,), lambda i: (i,)),
      ],
      out_shape=[
          jax.ShapeDtypeStruct((NZ // 4, D), jnp.float32),
          jax.ShapeDtypeStruct((NZ // 4, D), jnp.float32),
          jax.ShapeDtypeStruct((NZ // 4, D), jnp.float32),
          jax.ShapeDtypeStruct((NA,), jnp.float32),
          jax.ShapeDtypeStruct((NA,), jnp.float32),
          jax.ShapeDtypeStruct((NA,), jnp.float32),
      ],
      scratch_shapes=[pltpu.VMEM((rb, RU), jnp.float32),
                      pltpu.VMEM((rb, RU), jnp.float32),
                      pltpu.VMEM((rb, RU), jnp.float32)],
  )(h, w1cat, b1a, w2a, b2a)


def _rel_pump(z_hbm, idx_hbm, out_hbm, rel_base, spec,
              ibufs, rbufs, sbuf, gsems, ssem):
  _, arity, _, blk, nblk = spec
  ab = arity * blk
  b4 = blk // 4
  wb0 = _worker_id() * nblk

  def idx_load(b):
    off = pl.multiple_of((wb0 + b) * ab, 8)
    pltpu.sync_copy(idx_hbm.at[pl.ds(off, ab)],
                    ibufs[b % 2].at[pl.ds(0, ab)])

  def gather_start(b):
    return pltpu.async_copy(z_hbm.at[ibufs[b % 2].at[pl.ds(0, ab)]],
                            rbufs[b % 2].at[pl.ds(0, ab)], gsems[b % 2])

  def sum_block(b):
    rbuf = rbufs[b % 2]

    def body(p, _):
      for c in range(4):
        e = 4 * p + c
        for h in range(2):
          acc = rbuf[e, pl.ds(16 * h, 16)]
          for a in range(1, arity):
            acc = acc + rbuf[a * blk + e, pl.ds(16 * h, 16)]
          sbuf[p, pl.ds(32 * c + 16 * h, 16)] = acc
      return 0

    lax.fori_loop(0, b4, body, 0)

  def store_start(b):
    off = pl.multiple_of(rel_base + (wb0 + b) * b4, 8)
    return pltpu.async_copy(sbuf.at[pl.ds(0, b4)],
                            out_hbm.at[pl.ds(off, b4)], ssem)

  idx_load(0)
  gathers = {0: gather_start(0)}
  if nblk > 1:
    idx_load(1)
  store = None
  for b in range(nblk):
    gathers.pop(b).wait()
    if b + 1 < nblk:
      gathers[b + 1] = gather_start(b + 1)
    if b + 2 < nblk:
      idx_load(b + 2)
    if store is not None:
      store.wait()
    sum_block(b)
    store = store_start(b)
  store.wait()


REL_BASE4 = (0, BOND[2] * NW // 4, (BOND[2] + ANGLE[2]) * NW // 4)
NROWS4 = (BOND[2] + ANGLE[2] + TORSION[2]) * NW // 4


def _relations_body(zb, za, zt, ib, ia, it, sall,
                    i0, i1, r0, r1, sbuf, g0, g1, ssem):
  _rel_pump(zb, ib, sall, REL_BASE4[0], BOND,
            (i0, i1), (r0, r1), sbuf, (g0, g1), ssem)
  _rel_pump(za, ia, sall, REL_BASE4[1], ANGLE,
            (i0, i1), (r0, r1), sbuf, (g0, g1), ssem)
  _rel_pump(zt, it, sall, REL_BASE4[2], TORSION,
            (i0, i1), (r0, r1), sbuf, (g0, g1), ssem)


def _relations(zb, za, zt, idx_flat):
  out_type = [
      jax.ShapeDtypeStruct((NROWS4, D), jnp.float32),
  ]
  scratch = [
      pltpu.VMEM((MAXROWS,), jnp.int32),
      pltpu.VMEM((MAXROWS,), jnp.int32),
      pltpu.VMEM((MAXROWS, RU), jnp.float32),
      pltpu.VMEM((MAXROWS, RU), jnp.float32),
      pltpu.VMEM((MAXB4, D), jnp.float32),
      pltpu.SemaphoreType.DMA,
      pltpu.SemaphoreType.DMA,
      pltpu.SemaphoreType.DMA,
  ]
  fn = pl.kernel(_relations_body, out_type=out_type, mesh=_mesh,
                 scratch_types=scratch,
                 compiler_params=pltpu.CompilerParams(
                     use_tc_tiling_on_sc=False))
  return fn(zb, za, zt, *idx_flat)[0]


def _pair_loop(i0_hbm, i1_hbm, sig_hbm, eps_hbm, spec,
               ktbl_v, stbl_v, ia_v, ib_v, so_v, eo_v):
  _, chunk, blk, nblk = spec
  base = _worker_id() * chunk

  def group(si, _):
    o = pl.multiple_of(si * 16, 16)
    ia = ia_v[pl.ds(o, 16)]
    ib = ib_v[pl.ds(o, 16)]
    ka = plsc.load_gather(ktbl_v, [ia])
    kb = plsc.load_gather(ktbl_v, [ib])
    so_v[pl.ds(o, 16)] = (ka + kb) * 0.5
    ea = plsc.load_gather(stbl_v, [ia])
    eb = plsc.load_gather(stbl_v, [ib])
    eo_v[pl.ds(o, 16)] = ea * eb
    return 0

  for b in range(nblk):
    off = pl.multiple_of(base + b * blk, 8)
    pltpu.sync_copy(i0_hbm.at[pl.ds(off, blk)], ia_v)
    pltpu.sync_copy(i1_hbm.at[pl.ds(off, blk)], ib_v)
    lax.fori_loop(0, blk // 16, group, 0)
    pltpu.sync_copy(so_v, sig_hbm.at[pl.ds(off, blk)])
    pltpu.sync_copy(eo_v, eps_hbm.at[pl.ds(off, blk)])


def _pairs_body(ktbl, stbl, p140, p141, pnb0, pnb1,
                sig14, eps14, signb, epsnb,
                ktbl_v, stbl_v, ia_v, ib_v, so_v, eo_v):
  pltpu.sync_copy(ktbl, ktbl_v)
  pltpu.sync_copy(stbl, stbl_v)
  _pair_loop(p140, p141, sig14, eps14, P14,
             ktbl_v, stbl_v, ia_v, ib_v, so_v, eo_v)
  _pair_loop(pnb0, pnb1, signb, epsnb, PNB,
             ktbl_v, stbl_v, ia_v, ib_v, so_v, eo_v)


def _pairs(ktbl, stbl, p140, p141, pnb0, pnb1):
  out_type = [
      jax.ShapeDtypeStruct((P14[1] * NW,), jnp.float32),
      jax.ShapeDtypeStruct((P14[1] * NW,), jnp.float32),
      jax.ShapeDtypeStruct((PNB[1] * NW,), jnp.float32),
      jax.ShapeDtypeStruct((PNB[1] * NW,), jnp.float32),
  ]
  scratch = [
      pltpu.VMEM((NA,), jnp.float32),
      pltpu.VMEM((NA,), jnp.float32),
      pltpu.VMEM((PBLK,), jnp.int32),
      pltpu.VMEM((PBLK,), jnp.int32),
      pltpu.VMEM((PBLK,), jnp.float32),
      pltpu.VMEM((PBLK,), jnp.float32),
  ]
  fn = pl.kernel(_pairs_body, out_type=out_type, mesh=_mesh,
                 scratch_types=scratch,
                 compiler_params=pltpu.CompilerParams(
                     use_tc_tiling_on_sc=False, needs_layout_passes=False))
  return fn(ktbl, stbl, p140, p141, pnb0, pnb1)


RCB = 512


def _stage_c_body(s_ref, b1_ref, w_ref, b2_ref, k_ref, e_ref):
  t = jnp.tanh(s_ref[...] + b1_ref[0])
  o8 = (jnp.dot(t, w_ref[0], preferred_element_type=jnp.float32)
        + b2_ref[0])
  k_ref[...] = jnp.abs(o8[:, :4])
  e_ref[...] = jnp.abs(o8[:, 4:])


def _stage_c(rows4, b1s, w2s, b2s):
  nb = [REL_BASE4[1] // RCB, REL_BASE4[2] // RCB]

  def rel_of(i):
    return (i >= nb[0]).astype(jnp.int32) + (i >= nb[1]).astype(jnp.int32)

  b1rep = jnp.stack([jnp.tile(b, 4).reshape(1, D) for b in b1s])
  eye4 = jnp.eye(4, dtype=jnp.float32)
  wk8 = jnp.stack([
      jnp.concatenate([jnp.kron(eye4, w[:, 0:1]), jnp.kron(eye4, w[:, 1:2])],
                      axis=1) for w in w2s])
  b2rep = jnp.stack([jnp.concatenate([jnp.tile(b[0:1], 4),
                                      jnp.tile(b[1:2], 4)]).reshape(1, 8)
                     for b in b2s])
  k4, e4 = pl.pallas_call(
      _stage_c_body,
      grid=(NROWS4 // RCB,),
      in_specs=[
          pl.BlockSpec((RCB, D), lambda i: (i, 0)),
          pl.BlockSpec((1, 1, D), lambda i: (rel_of(i), 0, 0)),
          pl.BlockSpec((1, D, 8), lambda i: (rel_of(i), 0, 0)),
          pl.BlockSpec((1, 1, 8), lambda i: (rel_of(i), 0, 0)),
      ],
      out_specs=[
          pl.BlockSpec((RCB, 4), lambda i: (i, 0)),
          pl.BlockSpec((RCB, 4), lambda i: (i, 0)),
      ],
      out_shape=[
          jax.ShapeDtypeStruct((NROWS4, 4), jnp.float32),
          jax.ShapeDtypeStruct((NROWS4, 4), jnp.float32),
      ],
  )(rows4, b1rep, wk8, b2rep)
  return k4.reshape(-1), e4.reshape(-1)


def _flat_idx(atoms, spec):
  n, arity, chunk, blk, nblk = spec
  npad = chunk * NW
  cols = jnp.pad(atoms, ((0, npad - n), (0, 0)))
  arr = cols.T.reshape(arity, NW * nblk, blk)
  return arr.transpose(1, 0, 2).reshape(-1)


def _pad_cols(atoms, npad):
  n = atoms.shape[0]
  return [jnp.pad(atoms[:, a], (0, npad - n)) for a in range(atoms.shape[1])]


def kernel(h, bond_atoms, angle_atoms, torsion_atoms, one_four_atoms,
           nonbonded_atoms,
           W1_atom, b1_atom, W2_atom, b2_atom,
           W1_bond, b1_bond, W2_bond, b2_bond,
           W1_angle, b1_angle, W2_angle, b2_angle,
           W1_torsion, b1_torsion, W2_torsion, b2_torsion):
  w1cat = jnp.concatenate([W1_atom, W1_bond, W1_angle, W1_torsion], axis=1)
  zb, za, zt, k_atom, eq_atom, sq_eq = _stage_a(
      h, w1cat, b1_atom.reshape(1, RU), W2_atom, b2_atom.reshape(1, 2))

  idx_flat = (_flat_idx(bond_atoms, BOND),
              _flat_idx(angle_atoms, ANGLE),
              _flat_idx(torsion_atoms, TORSION))
  sall = _relations(zb.reshape(NZ, RU), za.reshape(NZ, RU),
                    zt.reshape(NZ, RU), idx_flat)

  p14 = _pad_cols(one_four_atoms, P14[1] * NW)
  pnb = _pad_cols(nonbonded_atoms, PNB[1] * NW)
  sig14, eps14, signb, epsnb = _pairs(k_atom, sq_eq, p14[0], p14[1],
                                      pnb[0], pnb[1])

  k1, e1 = _stage_c(sall, (b1_bond, b1_angle, b1_torsion),
                    (W2_bond, W2_angle, W2_torsion),
                    (b2_bond, b2_angle, b2_torsion))
  ob = REL_BASE4[1] * 4
  ot = REL_BASE4[2] * 4

  return jnp.concatenate([
      k_atom, eq_atom,
      k1[:BOND[0]], e1[:BOND[0]],
      k1[ob:ob + ANGLE[0]], e1[ob:ob + ANGLE[0]],
      k1[ot:ot + TORSION[0]], e1[ot:ot + TORSION[0]],
      sig14[:P14[0]], eps14[:P14[0]],
      signb[:PNB[0]], epsnb[:PNB[0]],
  ])
